# baseline (speedup 1.0000x reference)
import functools

import jax
import jax.numpy as jnp
from jax import lax
from jax.experimental import pallas as pl
from jax.experimental.pallas import tpu as pltpu
from jax.experimental.pallas import tpu_sc as plsc

NC = 2
NS = 16
NW = NC * NS
LANES = 16

N_PAIRS = 262144
PAIRS_PER_TILE = N_PAIRS // NW
ELEMS_PER_TILE = 2 * PAIRS_PER_TILE
CHUNK_ROWS = 256
CHUNK_PAIRS = CHUNK_ROWS // 2
N_CHUNKS = ELEMS_PER_TILE // CHUNK_ROWS
DMA_ROWS = 128
D = 64
NEG_INIT = -3.0e38


def _sc_body(table, pos_flat, neg_flat, part_out,
             idx_v, rows_v, part_v, sem0, sem1):
    c = lax.axis_index("c")
    s = lax.axis_index("s")
    wid = s * NC + c
    sems = (sem0, sem1)
    iota = lax.iota(jnp.int32, LANES)

    for stream, pairs_hbm in enumerate((pos_flat, neg_flat)):
        ebase = wid * ELEMS_PER_TILE
        pltpu.sync_copy(pairs_hbm.at[pl.ds(ebase, ELEMS_PER_TILE)], idx_v)

        def _tf(i, carry):
            for u in range(8):
                off = (i * 8 + u) * LANES
                v = idx_v[pl.ds(off, LANES)]
                idx_v[pl.ds(off, LANES)] = ((v & 16383) << 1) | (v >> 14)
            return carry
        lax.fori_loop(0, ELEMS_PER_TILE // (8 * LANES), _tf, 0)

        def _start(chunk, buf):
            for h in range(2):
                pltpu.make_async_copy(
                    table.at[idx_v.at[pl.ds(chunk * CHUNK_ROWS + h * DMA_ROWS,
                                            DMA_ROWS)]],
                    rows_v.at[buf, pl.ds(h * DMA_ROWS, DMA_ROWS)],
                    sems[buf]).start()

        def _wait(chunk, buf):
            for h in range(2):
                pltpu.make_async_copy(
                    table.at[idx_v.at[pl.ds(chunk * CHUNK_ROWS + h * DMA_ROWS,
                                            DMA_ROWS)]],
                    rows_v.at[buf, pl.ds(h * DMA_ROWS, DMA_ROWS)],
                    sems[buf]).wait()

        _start(0, 0)
        sign = 1.0 if stream == 0 else -1.0

        def _outer(o, ms):
            for b in range(2):
                g = o * 2 + b
                nxt = g + 1

                @pl.when(nxt < N_CHUNKS)
                def _():
                    _start(nxt, 1 - b)

                _wait(g, b)
                rows = rows_v.at[b]

                def _group(gl, ms2):
                    m, sacc = ms2
                    p0 = gl * LANES
                    dvec = jnp.zeros((LANES,), jnp.float32)
                    for j in range(LANES):
                        acc = jnp.zeros((LANES,), jnp.float32)
                        for k in range(D // (2 * LANES)):
                            av = rows[2 * (p0 + j), pl.ds(k * 2 * LANES,
                                                          2 * LANES)]
                            bv = rows[2 * (p0 + j) + 1, pl.ds(k * 2 * LANES,
                                                              2 * LANES)]
                            a0, a1 = plsc.unpack(
                                av, format=plsc.PackFormat.INTERLEAVED)
                            b0, b1 = plsc.unpack(
                                bv, format=plsc.PackFormat.INTERLEAVED)
                            d0 = a0 - b0
                            d1 = a1 - b1
                            acc = acc + d0 * d0 + d1 * d1
                        dvec = jnp.where(iota == j, jnp.sum(acc), dvec)
                    val = dvec * sign
                    m2 = jnp.maximum(m, val)
                    s2 = sacc * jnp.exp(m - m2) + jnp.exp(val - m2)
                    return (m2, s2)
                ms = lax.fori_loop(0, CHUNK_PAIRS // LANES, _group, ms)
            return ms

        init = (jnp.full((LANES,), NEG_INIT, jnp.float32),
                jnp.zeros((LANES,), jnp.float32))
        m_fin, s_fin = lax.fori_loop(0, N_CHUNKS // 2, _outer, init)
        part_v[pl.ds(stream * 2 * LANES, LANES)] = m_fin
        part_v[pl.ds(stream * 2 * LANES + LANES, LANES)] = s_fin

    pltpu.sync_copy(part_v, part_out.at[wid])


_sc_partials = functools.partial(
    pl.kernel,
    out_type=jax.ShapeDtypeStruct((NW, 4 * LANES), jnp.float32),
    mesh=plsc.VectorSubcoreMesh(core_axis_name="c", subcore_axis_name="s"),
    compiler_params=pltpu.CompilerParams(needs_layout_passes=False,
                                         use_tc_tiling_on_sc=False),
    scratch_types=(
        pltpu.VMEM((ELEMS_PER_TILE,), jnp.int32),
        pltpu.VMEM((2, CHUNK_ROWS, D), jnp.bfloat16),
        pltpu.VMEM((4 * LANES,), jnp.float32),
        pltpu.SemaphoreType.DMA,
        pltpu.SemaphoreType.DMA,
    ),
)(_sc_body)


def _loss_body(part_ref, out_ref):
    part = part_ref[...]
    mp = part[:, 0:LANES]
    sp = part[:, LANES:2 * LANES]
    mn = part[:, 2 * LANES:3 * LANES]
    sn = part[:, 3 * LANES:4 * LANES]
    logn = jnp.log(jnp.float32(N_PAIRS))
    big_mp = jnp.max(mp)
    lse_p = big_mp + jnp.log(jnp.sum(sp * jnp.exp(mp - big_mp)))
    big_mn = jnp.max(mn)
    lse_n = big_mn + jnp.log(jnp.sum(sn * jnp.exp(mn - big_mn)))
    out_ref[0, 0] = lse_p + lse_n - 2.0 * logn


_loss_reduce = pl.pallas_call(
    _loss_body,
    out_shape=jax.ShapeDtypeStruct((1, 1), jnp.float32),
    out_specs=pl.BlockSpec(memory_space=pltpu.SMEM),
)


def kernel(projections, pos_pairs, neg_pairs):
    table = projections.reshape(2 * projections.shape[0], D).astype(jnp.bfloat16)
    part = _sc_partials(table, pos_pairs.reshape(-1), neg_pairs.reshape(-1))
    loss = _loss_reduce(part)
    return loss[0, 0]


# speedup vs baseline: 3.2185x; 3.2185x over previous
import functools

import jax
import jax.numpy as jnp
from jax import lax
from jax.experimental import pallas as pl
from jax.experimental.pallas import tpu as pltpu
from jax.experimental.pallas import tpu_sc as plsc

NC = 2
NS = 16
NW = NC * NS
LANES = 16

N_PAIRS = 262144
PAIRS_PER_TILE = N_PAIRS // NW
ELEMS_PER_TILE = 2 * PAIRS_PER_TILE
CHUNK_ROWS = 256
CHUNK_PAIRS = CHUNK_ROWS // 2
N_CHUNKS = ELEMS_PER_TILE // CHUNK_ROWS
DMA_ROWS = 128
D = 64
NEG_INIT = -3.0e38


def _sc_body(table, pos_flat, neg_flat, part_out,
             idx_v, rows_v, part_v, sem0, sem1):
    c = lax.axis_index("c")
    s = lax.axis_index("s")
    wid = s * NC + c
    sems = (sem0, sem1)
    iota = lax.iota(jnp.int32, LANES)

    for stream, pairs_hbm in enumerate((pos_flat, neg_flat)):
        ebase = wid * ELEMS_PER_TILE
        pltpu.sync_copy(pairs_hbm.at[pl.ds(ebase, ELEMS_PER_TILE)], idx_v)

        def _tf(i, carry):
            for u in range(8):
                off = (i * 8 + u) * LANES
                v = idx_v[pl.ds(off, LANES)]
                idx_v[pl.ds(off, LANES)] = ((v & 16383) << 1) | (v >> 14)
            return carry
        lax.fori_loop(0, ELEMS_PER_TILE // (8 * LANES), _tf, 0)

        def _start(chunk, buf):
            for h in range(2):
                pltpu.make_async_copy(
                    table.at[idx_v.at[pl.ds(chunk * CHUNK_ROWS + h * DMA_ROWS,
                                            DMA_ROWS)]],
                    rows_v.at[buf, pl.ds(h * DMA_ROWS, DMA_ROWS)],
                    sems[buf]).start()

        def _wait(chunk, buf):
            for h in range(2):
                pltpu.make_async_copy(
                    table.at[idx_v.at[pl.ds(chunk * CHUNK_ROWS + h * DMA_ROWS,
                                            DMA_ROWS)]],
                    rows_v.at[buf, pl.ds(h * DMA_ROWS, DMA_ROWS)],
                    sems[buf]).wait()

        _start(0, 0)
        sign = 1.0 if stream == 0 else -1.0

        def _outer(o, ms):
            for b in range(2):
                g = o * 2 + b
                nxt = g + 1

                @pl.when(nxt < N_CHUNKS)
                def _():
                    _start(nxt, 1 - b)

                _wait(g, b)
                rows = rows_v.at[b]

                def _group(gl, ms2):
                    m, sacc = ms2
                    p0 = gl * LANES
                    dvec = jnp.zeros((LANES,), jnp.float32)
                    for j in range(LANES):
                        acc = jnp.zeros((LANES,), jnp.float32)
                        for k in range(D // (2 * LANES)):
                            av = rows[p0 + j, pl.ds(k * 2 * LANES, 2 * LANES)]
                            bv = rows[DMA_ROWS + p0 + j,
                                      pl.ds(k * 2 * LANES, 2 * LANES)]
                            dd = av - bv
                            d0, d1 = plsc.unpack(
                                dd, format=plsc.PackFormat.INTERLEAVED)
                            acc = acc + d0 * d0 + d1 * d1
                        dvec = jnp.where(iota == j, jnp.sum(acc), dvec)
                    val = dvec * sign
                    m2 = jnp.maximum(m, val)
                    s2 = sacc * jnp.exp(m - m2) + jnp.exp(val - m2)
                    return (m2, s2)
                ms = lax.fori_loop(0, CHUNK_PAIRS // LANES, _group, ms)
            return ms

        init = (jnp.full((LANES,), NEG_INIT, jnp.float32),
                jnp.zeros((LANES,), jnp.float32))
        m_fin, s_fin = lax.fori_loop(0, N_CHUNKS // 2, _outer, init)
        part_v[pl.ds(stream * 2 * LANES, LANES)] = m_fin
        part_v[pl.ds(stream * 2 * LANES + LANES, LANES)] = s_fin

    pltpu.sync_copy(part_v, part_out.at[wid])


_sc_partials = functools.partial(
    pl.kernel,
    out_type=jax.ShapeDtypeStruct((NW, 4 * LANES), jnp.float32),
    mesh=plsc.VectorSubcoreMesh(core_axis_name="c", subcore_axis_name="s"),
    compiler_params=pltpu.CompilerParams(needs_layout_passes=False,
                                         use_tc_tiling_on_sc=False),
    scratch_types=(
        pltpu.VMEM((ELEMS_PER_TILE,), jnp.int32),
        pltpu.VMEM((2, CHUNK_ROWS, D), jnp.bfloat16),
        pltpu.VMEM((4 * LANES,), jnp.float32),
        pltpu.SemaphoreType.DMA,
        pltpu.SemaphoreType.DMA,
    ),
)(_sc_body)


def _loss_body(part_ref, out_ref):
    part = part_ref[...]
    mp = part[:, 0:LANES]
    sp = part[:, LANES:2 * LANES]
    mn = part[:, 2 * LANES:3 * LANES]
    sn = part[:, 3 * LANES:4 * LANES]
    logn = jnp.log(jnp.float32(N_PAIRS))
    big_mp = jnp.max(mp)
    lse_p = big_mp + jnp.log(jnp.sum(sp * jnp.exp(mp - big_mp)))
    big_mn = jnp.max(mn)
    lse_n = big_mn + jnp.log(jnp.sum(sn * jnp.exp(mn - big_mn)))
    out_ref[0, 0] = lse_p + lse_n - 2.0 * logn


_loss_reduce = pl.pallas_call(
    _loss_body,
    out_shape=jax.ShapeDtypeStruct((1, 1), jnp.float32),
    out_specs=pl.BlockSpec(memory_space=pltpu.SMEM),
)


def kernel(projections, pos_pairs, neg_pairs):
    table = projections.reshape(2 * projections.shape[0], D)
    table = table.astype(jnp.bfloat16)
    def _phys(pairs):
        n = pairs.shape[0]
        return pairs.reshape(n // 128, 128, 2).transpose(0, 2, 1).reshape(-1)

    part = _sc_partials(table, _phys(pos_pairs), _phys(neg_pairs))
    loss = _loss_reduce(part)
    return loss[0, 0]


# speedup vs baseline: 3.2216x; 1.0009x over previous
import functools

import jax
import jax.numpy as jnp
from jax import lax
from jax.experimental import pallas as pl
from jax.experimental.pallas import tpu as pltpu
from jax.experimental.pallas import tpu_sc as plsc

NC = 2
NS = 16
NW = NC * NS
LANES = 16

N_PAIRS = 262144
PAIRS_PER_TILE = N_PAIRS // NW
ELEMS_PER_TILE = 2 * PAIRS_PER_TILE
CHUNK_ROWS = 256
CHUNK_PAIRS = CHUNK_ROWS // 2
N_CHUNKS = ELEMS_PER_TILE // CHUNK_ROWS
DMA_ROWS = 128
D = 64
NEG_INIT = -3.0e38


def _sc_body(table, pos_flat, neg_flat, part_out,
             idx_v, rows_v, part_v, sem0, sem1):
    c = lax.axis_index("c")
    s = lax.axis_index("s")
    wid = s * NC + c
    sems = (sem0, sem1)
    iota = lax.iota(jnp.int32, LANES)

    for stream, pairs_hbm in enumerate((pos_flat, neg_flat)):
        ebase = wid * ELEMS_PER_TILE
        pltpu.sync_copy(pairs_hbm.at[pl.ds(ebase, ELEMS_PER_TILE)], idx_v)

        def _tf_chunk(chunk):
            for u in range(CHUNK_ROWS // LANES):
                off = chunk * CHUNK_ROWS + u * LANES
                v = idx_v[pl.ds(off, LANES)]
                idx_v[pl.ds(off, LANES)] = ((v & 16383) << 1) | (v >> 14)

        def _start(chunk, buf):
            for h in range(2):
                pltpu.make_async_copy(
                    table.at[idx_v.at[pl.ds(chunk * CHUNK_ROWS + h * DMA_ROWS,
                                            DMA_ROWS)]],
                    rows_v.at[buf, pl.ds(h * DMA_ROWS, DMA_ROWS)],
                    sems[buf]).start()

        def _wait(chunk, buf):
            for h in range(2):
                pltpu.make_async_copy(
                    table.at[idx_v.at[pl.ds(chunk * CHUNK_ROWS + h * DMA_ROWS,
                                            DMA_ROWS)]],
                    rows_v.at[buf, pl.ds(h * DMA_ROWS, DMA_ROWS)],
                    sems[buf]).wait()

        _tf_chunk(0)
        _start(0, 0)
        sign = 1.0 if stream == 0 else -1.0

        def _outer(o, ms):
            for b in range(2):
                g = o * 2 + b
                nxt = g + 1

                @pl.when(nxt < N_CHUNKS)
                def _():
                    _tf_chunk(nxt)
                    _start(nxt, 1 - b)

                _wait(g, b)
                rows = rows_v.at[b]

                def _group(gl, ms2):
                    m, sacc = ms2
                    p0 = gl * LANES
                    dvec = jnp.zeros((LANES,), jnp.float32)
                    for j in range(LANES):
                        acc = jnp.zeros((LANES,), jnp.float32)
                        for k in range(D // (2 * LANES)):
                            av = rows[p0 + j, pl.ds(k * 2 * LANES, 2 * LANES)]
                            bv = rows[DMA_ROWS + p0 + j,
                                      pl.ds(k * 2 * LANES, 2 * LANES)]
                            dd = av - bv
                            d0, d1 = plsc.unpack(
                                dd, format=plsc.PackFormat.INTERLEAVED)
                            acc = acc + d0 * d0 + d1 * d1
                        dvec = jnp.where(iota == j, jnp.sum(acc), dvec)
                    val = dvec * sign
                    m2 = jnp.maximum(m, val)
                    s2 = sacc * jnp.exp(m - m2) + jnp.exp(val - m2)
                    return (m2, s2)
                ms = lax.fori_loop(0, CHUNK_PAIRS // LANES, _group, ms)
            return ms

        init = (jnp.full((LANES,), NEG_INIT, jnp.float32),
                jnp.zeros((LANES,), jnp.float32))
        m_fin, s_fin = lax.fori_loop(0, N_CHUNKS // 2, _outer, init)
        part_v[pl.ds(stream * 2 * LANES, LANES)] = m_fin
        part_v[pl.ds(stream * 2 * LANES + LANES, LANES)] = s_fin

    pltpu.sync_copy(part_v, part_out.at[wid])


_sc_partials = functools.partial(
    pl.kernel,
    out_type=jax.ShapeDtypeStruct((NW, 4 * LANES), jnp.float32),
    mesh=plsc.VectorSubcoreMesh(core_axis_name="c", subcore_axis_name="s"),
    compiler_params=pltpu.CompilerParams(needs_layout_passes=False,
                                         use_tc_tiling_on_sc=False),
    scratch_types=(
        pltpu.VMEM((ELEMS_PER_TILE,), jnp.int32),
        pltpu.VMEM((2, CHUNK_ROWS, D), jnp.bfloat16),
        pltpu.VMEM((4 * LANES,), jnp.float32),
        pltpu.SemaphoreType.DMA,
        pltpu.SemaphoreType.DMA,
    ),
)(_sc_body)


def _loss_body(part_ref, out_ref):
    part = part_ref[...]
    mp = part[:, 0:LANES]
    sp = part[:, LANES:2 * LANES]
    mn = part[:, 2 * LANES:3 * LANES]
    sn = part[:, 3 * LANES:4 * LANES]
    logn = jnp.log(jnp.float32(N_PAIRS))
    big_mp = jnp.max(mp)
    lse_p = big_mp + jnp.log(jnp.sum(sp * jnp.exp(mp - big_mp)))
    big_mn = jnp.max(mn)
    lse_n = big_mn + jnp.log(jnp.sum(sn * jnp.exp(mn - big_mn)))
    out_ref[0, 0] = lse_p + lse_n - 2.0 * logn


_loss_reduce = pl.pallas_call(
    _loss_body,
    out_shape=jax.ShapeDtypeStruct((1, 1), jnp.float32),
    out_specs=pl.BlockSpec(memory_space=pltpu.SMEM),
)


def kernel(projections, pos_pairs, neg_pairs):
    table = projections.reshape(2 * projections.shape[0], D)
    table = table.astype(jnp.bfloat16)
    def _phys(pairs):
        n = pairs.shape[0]
        return pairs.reshape(n // 128, 128, 2).transpose(0, 2, 1).reshape(-1)

    part = _sc_partials(table, _phys(pos_pairs), _phys(neg_pairs))
    loss = _loss_reduce(part)
    return loss[0, 0]
